# Initial kernel scaffold; baseline (speedup 1.0000x reference)
import jax
import jax.numpy as jnp
from jax.experimental import pallas as pl


def kernel(prev_h, emb_rel, edge_index, rid, pos_proj_w, attn_fc_w, weight_neighbor, k):
    raise NotImplementedError("write your pallas kernel here")


# speedup vs baseline: 2.1286x; 2.1286x over previous
import functools

import jax
import jax.numpy as jnp
from jax import lax
from jax.experimental import pallas as pl
from jax.experimental.pallas import tpu as pltpu
from jax.experimental.pallas import tpu_sc as plsc

N = 10000
E = 160000
D = 256
NREL = 200

NPAD = 10240
EPAD = 163840
RPAD = 208
K = 32

def _lane_iota():
    return lax.iota(jnp.int32, 16)


def _splat_sum(v):
    dnums = lax.GatherDimensionNumbers(
        offset_dims=(), collapsed_slice_dims=(0,), start_index_map=(0,))
    for sh in (8, 4, 2, 1):
        idx = jnp.bitwise_and(_lane_iota() + sh, 15)
        v = v + lax.gather(v, idx[:, None], dnums, (1,),
                           mode=lax.GatherScatterMode.PROMISE_IN_BOUNDS)
    return v


def _mm_body(x_ref, w_ref, o_ref):
    o_ref[...] = jnp.dot(x_ref[...], w_ref[...],
                         preferred_element_type=jnp.float32)


def _tc_matmul(x, w, block_rows):
    m, kdim = x.shape
    n = w.shape[1]
    grid = (m // block_rows,)
    return pl.pallas_call(
        _mm_body,
        grid=grid,
        in_specs=[
            pl.BlockSpec((block_rows, kdim), lambda i: (i, 0)),
            pl.BlockSpec((kdim, n), lambda i: (0, 0)),
        ],
        out_specs=pl.BlockSpec((block_rows, n), lambda i: (i, 0)),
        out_shape=jax.ShapeDtypeStruct((m, n), jnp.float32),
    )(x, w)


def _pass_a_body(srcp, dstp, p1, p2, wvec, ex_out, d2_out,
                 srcv, dstv, rows1, rows2, exb, wv, denom_sp, sem1, sem2):
    cid = lax.axis_index("c")
    sid = lax.axis_index("s")
    wid = cid * 16 + sid

    pltpu.sync_copy(wvec, wv)
    zv = jnp.zeros((16,), jnp.float32)
    for g in range(K // 16):
        exb[pl.ds(g * 16, 16)] = zv
    @pl.loop(0, NPAD // 16 // K)
    def _zero(t):
        pltpu.sync_copy(exb, denom_sp.at[pl.ds(sid * (NPAD // 16) + t * K, K)])
    plsc.subcore_barrier()

    wvs = [wv[pl.ds(c * 16, 16)] for c in range(16)]
    base0 = wid * (EPAD // 32)

    @pl.loop(0, EPAD // 32 // K)
    def _chunk(ci):
        base = base0 + ci * K
        pltpu.sync_copy(srcp.at[pl.ds(base, K)], srcv)
        pltpu.sync_copy(dstp.at[pl.ds(base, K)], dstv)
        c1 = pltpu.async_copy(p1.at[srcv], rows1, sem1)
        c2 = pltpu.async_copy(p2.at[dstv], rows2, sem2)
        c1.wait()
        c2.wait()
        for g in range(K // 16):
            def edge(jj, ev):
                j = g * 16 + jj
                acc = jnp.zeros((16,), jnp.float32)
                for c in range(16):
                    v = rows1[j, pl.ds(c * 16, 16)] + rows2[j, pl.ds(c * 16, 16)]
                    acc = acc + jnp.maximum(v, 0.01 * v) * wvs[c]
                ejs = _splat_sum(acc)
                return jnp.where(_lane_iota() == jj, ejs, ev)
            evec = lax.fori_loop(0, 16, edge, jnp.zeros((16,), jnp.float32))
            exb[pl.ds(g * 16, 16)] = jnp.exp(evec)
        pltpu.sync_copy(exb, ex_out.at[pl.ds(base, K)])
        pltpu.sync_copy(exb, denom_sp.at[dstv], add=True)

    plsc.subcore_barrier()

    @pl.when(sid == 0)
    def _():
        pltpu.sync_copy(denom_sp, d2_out.at[cid])


def _pass_b_body(srcp, dstp, ridp, ex_in, qlo, qhi, qrcat, a2_out,
                 srcv, dstv, ridv, exv, rowsq, stage, qr_v, a_sp, semg):
    cid = lax.axis_index("c")
    sid = lax.axis_index("s")

    pltpu.sync_copy(qrcat.at[pl.ds(cid * RPAD, RPAD)], qr_v)

    zv = jnp.zeros((16,), jnp.float32)
    for j in range(K):
        for c in range(8):
            stage[j, pl.ds(c * 16, 16)] = zv
    @pl.loop(0, NPAD // 16 // K)
    def _zero(t):
        pltpu.sync_copy(stage, a_sp.at[pl.ds(sid * (NPAD // 16) + t * K, K)])
    plsc.subcore_barrier()

    base0 = sid * (EPAD // 16)

    @pl.loop(0, EPAD // 16 // K)
    def _chunk(ci):
        base = base0 + ci * K
        pltpu.sync_copy(srcp.at[pl.ds(base, K)], srcv)
        pltpu.sync_copy(dstp.at[pl.ds(base, K)], dstv)
        pltpu.sync_copy(ridp.at[pl.ds(base, K)], ridv)
        pltpu.sync_copy(ex_in.at[pl.ds(base, K)], exv)

        @pl.when(cid == 0)
        def _():
            pltpu.async_copy(qlo.at[srcv], rowsq, semg).wait()

        @pl.when(cid == 1)
        def _():
            pltpu.async_copy(qhi.at[srcv], rowsq, semg).wait()

        @pl.loop(0, K)
        def _edge(j):
            jsplat = jnp.full((16,), j, dtype=jnp.int32)
            exs = plsc.load_gather(exv, [jsplat])
            rids = plsc.load_gather(ridv, [jsplat])
            for c in range(8):
                q = plsc.load_gather(qr_v, [rids, _lane_iota() + c * 16])
                stage[j, pl.ds(c * 16, 16)] = \
                    (rowsq[j, pl.ds(c * 16, 16)] + q) * exs

        pltpu.sync_copy(stage, a_sp.at[dstv], add=True)

    plsc.subcore_barrier()
    pltpu.sync_copy(a_sp.at[pl.ds(sid * (NPAD // 16), NPAD // 16)],
                    a2_out.at[cid, pl.ds(sid * (NPAD // 16), NPAD // 16)])


def _final_body(a0_ref, a1_ref, d0_ref, d1_ref, o_ref):
    d = d0_ref[...] + d1_ref[...]
    ok = d > 0.0
    safe = jnp.where(ok, d, 1.0)
    o_ref[:, :128] = jnp.where(ok, a0_ref[...] / safe, 0.0)
    o_ref[:, 128:] = jnp.where(ok, a1_ref[...] / safe, 0.0)


def kernel(prev_h, emb_rel, edge_index, rid, pos_proj_w, attn_fc_w,
           weight_neighbor, k):
    f32 = jnp.float32
    i32 = jnp.int32

    wa = pos_proj_w[:D]
    wb = pos_proj_w[D:]
    wna = weight_neighbor[:D]
    wnb = weight_neighbor[D:]
    wcat = jnp.concatenate([wa, wb, wna], axis=1)

    pq = _tc_matmul(prev_h.astype(f32), wcat.astype(f32), 1000)
    qr = _tc_matmul(emb_rel.astype(f32), wnb.astype(f32), 200)

    zrows = jnp.zeros((NPAD - N, 128), f32)
    p1 = jnp.concatenate([pq[:, :D], jnp.zeros((NPAD - N, D), f32)], axis=0)
    p2 = jnp.concatenate([pq[:, D:2 * D], jnp.zeros((NPAD - N, D), f32)],
                         axis=0)
    qlo = jnp.concatenate([pq[:, 2 * D:2 * D + 128], zrows], axis=0)
    qhi = jnp.concatenate([pq[:, 2 * D + 128:], zrows], axis=0)
    zr = jnp.zeros((RPAD - NREL, 128), f32)
    qrcat = jnp.concatenate([qr[:, :128], zr, qr[:, 128:], zr], axis=0)

    pad = EPAD - E
    sent_n = (N + (jnp.arange(pad, dtype=i32) % 8)).astype(i32)
    sent_r = (NREL + (jnp.arange(pad, dtype=i32) % 8)).astype(i32)
    srcp = jnp.concatenate([edge_index[0].astype(i32), sent_n])
    dstp = jnp.concatenate([edge_index[1].astype(i32), sent_n])
    ridp = jnp.concatenate([rid.astype(i32), sent_r])
    wvec = attn_fc_w[:, 0].astype(f32)

    mesh = plsc.VectorSubcoreMesh(core_axis_name="c", subcore_axis_name="s")

    sc_params = pltpu.CompilerParams(needs_layout_passes=False)

    pass_a = functools.partial(
        pl.kernel,
        mesh=mesh,
        compiler_params=sc_params,
        out_type=[
            jax.ShapeDtypeStruct((EPAD,), f32),
            jax.ShapeDtypeStruct((2, NPAD), f32),
        ],
        scratch_types=[
            pltpu.VMEM((K,), i32),
            pltpu.VMEM((K,), i32),
            pltpu.VMEM((K, D), f32),
            pltpu.VMEM((K, D), f32),
            pltpu.VMEM((K,), f32),
            pltpu.VMEM((D,), f32),
            pltpu.VMEM_SHARED((NPAD,), f32),
            pltpu.SemaphoreType.DMA,
            pltpu.SemaphoreType.DMA,
        ],
    )(_pass_a_body)
    ex, d2 = pass_a(srcp, dstp, p1, p2, wvec)

    pass_b = functools.partial(
        pl.kernel,
        mesh=mesh,
        compiler_params=sc_params,
        out_type=jax.ShapeDtypeStruct((2, NPAD, 128), f32),
        scratch_types=[
            pltpu.VMEM((K,), i32),
            pltpu.VMEM((K,), i32),
            pltpu.VMEM((K,), i32),
            pltpu.VMEM((K,), f32),
            pltpu.VMEM((K, 128), f32),
            pltpu.VMEM((K, 128), f32),
            pltpu.VMEM((RPAD, 128), f32),
            pltpu.VMEM_SHARED((NPAD, 128), f32),
            pltpu.SemaphoreType.DMA,
        ],
    )(_pass_b_body)
    a2 = pass_b(srcp, dstp, ridp, ex, qlo, qhi, qrcat)

    out = pl.pallas_call(
        _final_body,
        grid=(8,),
        in_specs=[
            pl.BlockSpec((NPAD // 8, 128), lambda i: (i, 0)),
            pl.BlockSpec((NPAD // 8, 128), lambda i: (i, 0)),
            pl.BlockSpec((NPAD // 8, 1), lambda i: (i, 0)),
            pl.BlockSpec((NPAD // 8, 1), lambda i: (i, 0)),
        ],
        out_specs=pl.BlockSpec((NPAD // 8, D), lambda i: (i, 0)),
        out_shape=jax.ShapeDtypeStruct((NPAD, D), f32),
    )(a2[0], a2[1], d2[0].reshape(NPAD, 1), d2[1].reshape(NPAD, 1))

    return out[:N]


# speedup vs baseline: 4.1426x; 1.9461x over previous
import functools

import jax
import jax.numpy as jnp
from jax import lax
from jax.experimental import pallas as pl
from jax.experimental.pallas import tpu as pltpu
from jax.experimental.pallas import tpu_sc as plsc

N = 10000
E = 160000
D = 256
NREL = 200

NPAD = 10240
EPAD = 163840
RPAD = 208
KA = 64
KB = 32
NCA = EPAD // 32 // KA
NCB = EPAD // 16 // KB


def _lane_iota():
    return lax.iota(jnp.int32, 16)


def _splat_sum(v):
    dnums = lax.GatherDimensionNumbers(
        offset_dims=(), collapsed_slice_dims=(0,), start_index_map=(0,))
    for sh in (8, 4, 2, 1):
        idx = jnp.bitwise_and(_lane_iota() + sh, 15)
        v = v + lax.gather(v, idx[:, None], dnums, (1,),
                           mode=lax.GatherScatterMode.PROMISE_IN_BOUNDS)
    return v


def _mm_body(x_ref, w_ref, o_ref):
    o_ref[...] = jnp.dot(x_ref[...], w_ref[...],
                         preferred_element_type=jnp.float32)


def _tc_matmul(x, w, block_rows):
    m, kdim = x.shape
    n = w.shape[1]
    grid = (m // block_rows,)
    return pl.pallas_call(
        _mm_body,
        grid=grid,
        in_specs=[
            pl.BlockSpec((block_rows, kdim), lambda i: (i, 0)),
            pl.BlockSpec((kdim, n), lambda i: (0, 0)),
        ],
        out_specs=pl.BlockSpec((block_rows, n), lambda i: (i, 0)),
        out_shape=jax.ShapeDtypeStruct((m, n), jnp.float32),
    )(x, w)


def _pass_a_body(srcp, dstp, p1, p2, wvec, ex2_out, d2_out,
                 srcb0, srcb1, dstb0, dstb1, rows1, rows2, exloc, exsb0,
                 exsb1, wv, zbuf, denom_sp,
                 si0, si1, g1a, g1b, g2a, g2b):
    cid = lax.axis_index("c")
    sid = lax.axis_index("s")
    wid = cid * 16 + sid

    pltpu.sync_copy(wvec, wv)

    zv = jnp.zeros((16,), jnp.float32)
    for g in range(640 // 16):
        zbuf[pl.ds(g * 16, 16)] = zv
    pltpu.sync_copy(zbuf, denom_sp.at[pl.ds(sid * 640, 640)])
    plsc.subcore_barrier()

    wvs = [wv[pl.ds(c * 16, 16)] for c in range(16)]
    srcb = (srcb0, srcb1)
    dstb = (dstb0, dstb1)
    exsb = (exsb0, exsb1)
    isems = (si0, si1)
    gsems = ((g1a, g2a), (g1b, g2b))
    ebase = wid * (EPAD // 32)

    def idx_copies(b, cc):
        return (pltpu.make_async_copy(
                    srcp.at[pl.ds(ebase + cc * KA, KA)], srcb[b], isems[b]),
                pltpu.make_async_copy(
                    dstp.at[pl.ds(ebase + cc * KA, KA)], dstb[b], isems[b]))

    def row_copies(b):
        return (pltpu.make_async_copy(p1.at[srcb[b]], rows1.at[b],
                                      gsems[b][0]),
                pltpu.make_async_copy(p2.at[dstb[b]], rows2.at[b],
                                      gsems[b][1]))

    for c0, c1 in (idx_copies(0, 0), idx_copies(1, 1)):
        c0.start()
        c1.start()
    for c in idx_copies(0, 0):
        c.wait()
    for c in row_copies(0):
        c.start()

    @pl.loop(0, NCA, step=2)
    def _chunk(c):
        for b in range(2):
            cc = c + b
            nb = 1 - b
            for cp in row_copies(b):
                cp.wait()

            @pl.when(cc + 1 < NCA)
            def _():
                for cp in idx_copies(nb, cc + 1):
                    cp.wait()
                for cp in row_copies(nb):
                    cp.start()

            for g in range(KA // 16):
                def edge(jj, ev):
                    j = g * 16 + jj
                    acc = jnp.zeros((16,), jnp.float32)
                    for cq in range(16):
                        v = (rows1[b, j, pl.ds(cq * 16, 16)]
                             + rows2[b, j, pl.ds(cq * 16, 16)])
                        acc = acc + jnp.maximum(v, 0.01 * v) * wvs[cq]
                    return jnp.where(_lane_iota() == jj, _splat_sum(acc), ev)
                evec = lax.fori_loop(0, 16, edge,
                                     jnp.zeros((16,), jnp.float32))
                exv = jnp.exp(evec)
                exloc[cc, pl.ds(g * 16, 16)] = exv
                exsb[b][pl.ds(g * 16, 16)] = exv
            pltpu.sync_copy(exsb[b], denom_sp.at[dstb[b]], add=True)

            @pl.when(cc + 2 < NCA)
            def _():
                for cp in idx_copies(b, cc + 2):
                    cp.start()

    pltpu.sync_copy(exloc, ex2_out.at[pl.ds(wid * NCA, NCA)])
    plsc.subcore_barrier()

    @pl.when(sid == 0)
    def _():
        pltpu.sync_copy(denom_sp, d2_out.at[cid])


def _pass_b_body(srcp, dstp, ridp, exp_, qlo, qhi, qrcat, a2_out,
                 srcb0, srcb1, dstb0, dstb1, ridb0, ridb1, exb0, exb1,
                 rowsq, qrrows, stage, qr_sp, a_sp,
                 si0, si1, sq0, sq1, sr0, sr1):
    cid = lax.axis_index("c")
    sid = lax.axis_index("s")

    @pl.when(sid == 0)
    def _():
        pltpu.sync_copy(qrcat.at[pl.ds(cid * RPAD, RPAD)], qr_sp)

    zv = jnp.zeros((16,), jnp.float32)
    for j in range(KB):
        for cq in range(8):
            stage[0, j, pl.ds(cq * 16, 16)] = zv
    @pl.loop(0, 640 // KB)
    def _zero(t):
        pltpu.sync_copy(stage.at[0], a_sp.at[pl.ds(sid * 640 + t * KB, KB)])
    plsc.subcore_barrier()

    srcb = (srcb0, srcb1)
    dstb = (dstb0, dstb1)
    ridb = (ridb0, ridb1)
    exb = (exb0, exb1)
    isems = (si0, si1)
    qsems = (sq0, sq1)
    rsems = (sr0, sr1)
    ebase = sid * (EPAD // 16)

    def idx_copies(b, cc):
        sl = pl.ds(ebase + cc * KB, KB)
        return (pltpu.make_async_copy(srcp.at[sl], srcb[b], isems[b]),
                pltpu.make_async_copy(dstp.at[sl], dstb[b], isems[b]),
                pltpu.make_async_copy(ridp.at[sl], ridb[b], isems[b]),
                pltpu.make_async_copy(exp_.at[sl], exb[b], isems[b]))

    def qr_copy(b):
        return pltpu.make_async_copy(qr_sp.at[ridb[b]], qrrows.at[b],
                                     rsems[b])

    def qs_start(b):
        @pl.when(cid == 0)
        def _():
            pltpu.make_async_copy(qlo.at[srcb[b]], rowsq.at[b],
                                  qsems[b]).start()

        @pl.when(cid == 1)
        def _():
            pltpu.make_async_copy(qhi.at[srcb[b]], rowsq.at[b],
                                  qsems[b]).start()

    def qs_wait(b):
        pltpu.make_async_copy(qlo.at[srcb[b]], rowsq.at[b], qsems[b]).wait()

    for cp in idx_copies(0, 0) + idx_copies(1, 1):
        cp.start()
    for cp in idx_copies(0, 0):
        cp.wait()
    qs_start(0)
    qr_copy(0).start()

    @pl.loop(0, NCB, step=2)
    def _chunk(c):
        for b in range(2):
            cc = c + b
            nb = 1 - b
            qs_wait(b)
            qr_copy(b).wait()

            @pl.when(cc + 1 < NCB)
            def _():
                for cp in idx_copies(nb, cc + 1):
                    cp.wait()
                qs_start(nb)
                qr_copy(nb).start()

            @pl.loop(0, KB, unroll=4)
            def _edge(j):
                js = jnp.full((16,), j, dtype=jnp.int32)
                exs = plsc.load_gather(exb[b], [js])
                for cq in range(8):
                    stage[b, j, pl.ds(cq * 16, 16)] = \
                        (rowsq[b, j, pl.ds(cq * 16, 16)]
                         + qrrows[b, j, pl.ds(cq * 16, 16)]) * exs

            pltpu.sync_copy(stage.at[b], a_sp.at[dstb[b]], add=True)

            @pl.when(cc + 2 < NCB)
            def _():
                for cp in idx_copies(b, cc + 2):
                    cp.start()

    plsc.subcore_barrier()
    pltpu.sync_copy(a_sp.at[pl.ds(sid * 640, 640)],
                    a2_out.at[cid, pl.ds(sid * 640, 640)])


def _final_body(a0_ref, a1_ref, d0_ref, d1_ref, o_ref):
    d = d0_ref[...] + d1_ref[...]
    ok = d > 0.0
    safe = jnp.where(ok, d, 1.0)
    o_ref[:, :128] = jnp.where(ok, a0_ref[...] / safe, 0.0)
    o_ref[:, 128:] = jnp.where(ok, a1_ref[...] / safe, 0.0)


def kernel(prev_h, emb_rel, edge_index, rid, pos_proj_w, attn_fc_w,
           weight_neighbor, k):
    f32 = jnp.float32
    i32 = jnp.int32

    wa = pos_proj_w[:D]
    wb = pos_proj_w[D:]
    wna = weight_neighbor[:D]
    wnb = weight_neighbor[D:]
    wcat = jnp.concatenate([wa, wb, wna], axis=1)

    pq = _tc_matmul(prev_h.astype(f32), wcat.astype(f32), 1000)
    qr = _tc_matmul(emb_rel.astype(f32), wnb.astype(f32), 200)

    zrows = jnp.zeros((NPAD - N, 128), f32)
    p1 = jnp.concatenate([pq[:, :D], jnp.zeros((NPAD - N, D), f32)], axis=0)
    p2 = jnp.concatenate([pq[:, D:2 * D], jnp.zeros((NPAD - N, D), f32)],
                         axis=0)
    qlo = jnp.concatenate([pq[:, 2 * D:2 * D + 128], zrows], axis=0)
    qhi = jnp.concatenate([pq[:, 2 * D + 128:], zrows], axis=0)
    zr = jnp.zeros((RPAD - NREL, 128), f32)
    qrcat = jnp.concatenate([qr[:, :128], zr, qr[:, 128:], zr], axis=0)

    pad = EPAD - E
    sent_n = (N + (jnp.arange(pad, dtype=i32) % 8)).astype(i32)
    sent_r = (NREL + (jnp.arange(pad, dtype=i32) % 8)).astype(i32)
    srcp = jnp.concatenate([edge_index[0].astype(i32), sent_n])
    dstp = jnp.concatenate([edge_index[1].astype(i32), sent_n])
    ridp = jnp.concatenate([rid.astype(i32), sent_r])
    wvec = attn_fc_w[:, 0].astype(f32)

    mesh = plsc.VectorSubcoreMesh(core_axis_name="c", subcore_axis_name="s")
    sc_params = pltpu.CompilerParams(needs_layout_passes=False)

    pass_a = functools.partial(
        pl.kernel,
        mesh=mesh,
        compiler_params=sc_params,
        out_type=[
            jax.ShapeDtypeStruct((EPAD // KA, KA), f32),
            jax.ShapeDtypeStruct((2, NPAD), f32),
        ],
        scratch_types=[
            pltpu.VMEM((KA,), i32),
            pltpu.VMEM((KA,), i32),
            pltpu.VMEM((KA,), i32),
            pltpu.VMEM((KA,), i32),
            pltpu.VMEM((2, KA, D), f32),
            pltpu.VMEM((2, KA, D), f32),
            pltpu.VMEM((NCA, KA), f32),
            pltpu.VMEM((KA,), f32),
            pltpu.VMEM((KA,), f32),
            pltpu.VMEM((D,), f32),
            pltpu.VMEM((640,), f32),
            pltpu.VMEM_SHARED((NPAD,), f32),
            pltpu.SemaphoreType.DMA,
            pltpu.SemaphoreType.DMA,
            pltpu.SemaphoreType.DMA,
            pltpu.SemaphoreType.DMA,
            pltpu.SemaphoreType.DMA,
            pltpu.SemaphoreType.DMA,
        ],
    )(_pass_a_body)
    ex2, d2 = pass_a(srcp, dstp, p1, p2, wvec)
    exflat = ex2.reshape(EPAD)

    pass_b = functools.partial(
        pl.kernel,
        mesh=mesh,
        compiler_params=sc_params,
        out_type=jax.ShapeDtypeStruct((2, NPAD, 128), f32),
        scratch_types=[
            pltpu.VMEM((KB,), i32),
            pltpu.VMEM((KB,), i32),
            pltpu.VMEM((KB,), i32),
            pltpu.VMEM((KB,), i32),
            pltpu.VMEM((KB,), i32),
            pltpu.VMEM((KB,), i32),
            pltpu.VMEM((KB,), f32),
            pltpu.VMEM((KB,), f32),
            pltpu.VMEM((2, KB, 128), f32),
            pltpu.VMEM((2, KB, 128), f32),
            pltpu.VMEM((2, KB, 128), f32),
            pltpu.VMEM_SHARED((RPAD, 128), f32),
            pltpu.VMEM_SHARED((NPAD, 128), f32),
            pltpu.SemaphoreType.DMA,
            pltpu.SemaphoreType.DMA,
            pltpu.SemaphoreType.DMA,
            pltpu.SemaphoreType.DMA,
            pltpu.SemaphoreType.DMA,
            pltpu.SemaphoreType.DMA,
        ],
    )(_pass_b_body)
    a2 = pass_b(srcp, dstp, ridp, exflat, qlo, qhi, qrcat)

    out = pl.pallas_call(
        _final_body,
        grid=(8,),
        in_specs=[
            pl.BlockSpec((NPAD // 8, 128), lambda i: (i, 0)),
            pl.BlockSpec((NPAD // 8, 128), lambda i: (i, 0)),
            pl.BlockSpec((NPAD // 8, 1), lambda i: (i, 0)),
            pl.BlockSpec((NPAD // 8, 1), lambda i: (i, 0)),
        ],
        out_specs=pl.BlockSpec((NPAD // 8, D), lambda i: (i, 0)),
        out_shape=jax.ShapeDtypeStruct((NPAD, D), f32),
    )(a2[0], a2[1], d2[0].reshape(NPAD, 1), d2[1].reshape(NPAD, 1))

    return out[:N]


# speedup vs baseline: 5.0726x; 1.2245x over previous
import functools

import jax
import jax.numpy as jnp
from jax import lax
from jax.experimental import pallas as pl
from jax.experimental.pallas import tpu as pltpu
from jax.experimental.pallas import tpu_sc as plsc

N = 10000
E = 160000
D = 256
NREL = 200

NPAD = 10240
EPAD = 163840
RPAD = 208
KA = 64
KB = 64
NCA = EPAD // 32 // KA
NCB = EPAD // 16 // KB


def _lane_iota():
    return lax.iota(jnp.int32, 16)


def _splat_sum(v):
    dnums = lax.GatherDimensionNumbers(
        offset_dims=(), collapsed_slice_dims=(0,), start_index_map=(0,))
    for sh in (8, 4, 2, 1):
        idx = jnp.bitwise_and(_lane_iota() + sh, 15)
        v = v + lax.gather(v, idx[:, None], dnums, (1,),
                           mode=lax.GatherScatterMode.PROMISE_IN_BOUNDS)
    return v


def _mm_body(x_ref, w_ref, o_ref):
    o_ref[...] = jnp.dot(x_ref[...], w_ref[...],
                         preferred_element_type=jnp.float32)


def _tc_matmul(x, w, block_rows):
    m, kdim = x.shape
    n = w.shape[1]
    grid = (m // block_rows,)
    return pl.pallas_call(
        _mm_body,
        grid=grid,
        in_specs=[
            pl.BlockSpec((block_rows, kdim), lambda i: (i, 0)),
            pl.BlockSpec((kdim, n), lambda i: (0, 0)),
        ],
        out_specs=pl.BlockSpec((block_rows, n), lambda i: (i, 0)),
        out_shape=jax.ShapeDtypeStruct((m, n), jnp.float32),
    )(x, w)


def _pass_a_body(srcp, dstp, p1, p2, wvec, ex2_out, d2_out,
                 srcb0, srcb1, dstb0, dstb1, dsca0, dsca1, rows1, rows2,
                 exloc, exsb0, exsb1, wv, zbuf, denom_sp,
                 si0, si1, g1a, g1b, g2a, g2b, ssa0, ssa1):
    cid = lax.axis_index("c")
    sid = lax.axis_index("s")
    wid = cid * 16 + sid

    pltpu.sync_copy(wvec, wv)

    zv = jnp.zeros((16,), jnp.float32)
    for g in range(640 // 16):
        zbuf[pl.ds(g * 16, 16)] = zv
    pltpu.sync_copy(zbuf, denom_sp.at[pl.ds(sid * 640, 640)])
    plsc.subcore_barrier()

    wvs = [wv[pl.ds(c * 16, 16)] for c in range(16)]
    srcb = (srcb0, srcb1)
    dstb = (dstb0, dstb1)
    dsca = (dsca0, dsca1)
    exsb = (exsb0, exsb1)
    isems = (si0, si1)
    gsems = ((g1a, g2a), (g1b, g2b))
    ssems = (ssa0, ssa1)
    ebase = wid * (EPAD // 32)

    def scat_copy(b):
        return pltpu.make_async_copy(exsb[b], denom_sp.at[dsca[b]], ssems[b])

    def idx_copies(b, cc):
        return (pltpu.make_async_copy(
                    srcp.at[pl.ds(ebase + cc * KA, KA)], srcb[b], isems[b]),
                pltpu.make_async_copy(
                    dstp.at[pl.ds(ebase + cc * KA, KA)], dstb[b], isems[b]))

    def row_copies(b):
        return (pltpu.make_async_copy(p1.at[srcb[b]], rows1.at[b],
                                      gsems[b][0]),
                pltpu.make_async_copy(p2.at[dstb[b]], rows2.at[b],
                                      gsems[b][1]))

    for c0, c1 in (idx_copies(0, 0), idx_copies(1, 1)):
        c0.start()
        c1.start()
    for c in idx_copies(0, 0):
        c.wait()
    for c in row_copies(0):
        c.start()

    @pl.loop(0, NCA, step=2)
    def _chunk(c):
        for b in range(2):
            cc = c + b
            nb = 1 - b
            for cp in row_copies(b):
                cp.wait()

            @pl.when(cc + 1 < NCA)
            def _():
                for cp in idx_copies(nb, cc + 1):
                    cp.wait()
                for cp in row_copies(nb):
                    cp.start()

            @pl.when(cc >= 2)
            def _():
                scat_copy(b).wait()

            for g in range(KA // 16):
                def edge(jj, ev):
                    j = g * 16 + jj
                    acc = jnp.zeros((16,), jnp.float32)
                    for cq in range(16):
                        v = (rows1[b, j, pl.ds(cq * 16, 16)]
                             + rows2[b, j, pl.ds(cq * 16, 16)])
                        acc = acc + jnp.maximum(v, 0.01 * v) * wvs[cq]
                    return jnp.where(_lane_iota() == jj, _splat_sum(acc), ev)
                evec = lax.fori_loop(0, 16, edge,
                                     jnp.zeros((16,), jnp.float32))
                exv = jnp.exp(evec)
                exloc[cc, pl.ds(g * 16, 16)] = exv
                exsb[b][pl.ds(g * 16, 16)] = exv
            for g in range(KA // 16):
                dsca[b][pl.ds(g * 16, 16)] = dstb[b][pl.ds(g * 16, 16)]
            scat_copy(b).start(add=True)

            @pl.when(cc + 2 < NCA)
            def _():
                for cp in idx_copies(b, cc + 2):
                    cp.start()

    scat_copy(0).wait()
    scat_copy(1).wait()
    pltpu.sync_copy(exloc, ex2_out.at[pl.ds(wid * NCA, NCA)])
    plsc.subcore_barrier()

    @pl.when(sid == 0)
    def _():
        pltpu.sync_copy(denom_sp, d2_out.at[cid])


def _pass_b_body(srcp, dstp, ridp, exp_, qlo, qhi, qrcat, a2_out,
                 srcb0, srcb1, dstb0, dstb1, ridb0, ridb1, exb0, exb1,
                 dsc0, dsc1, rowsq, stage, qr_sp, a_sp,
                 si0, si1, sq0, sq1, sr0, sr1, ss0, ss1):
    cid = lax.axis_index("c")
    sid = lax.axis_index("s")

    @pl.when(sid == 0)
    def _():
        pltpu.sync_copy(qrcat.at[pl.ds(cid * RPAD, RPAD)], qr_sp)

    zv = jnp.zeros((16,), jnp.float32)
    for j in range(KB):
        for cq in range(8):
            stage[0, j, pl.ds(cq * 16, 16)] = zv
    @pl.loop(0, 640 // KB)
    def _zero(t):
        pltpu.sync_copy(stage.at[0], a_sp.at[pl.ds(sid * 640 + t * KB, KB)])
    plsc.subcore_barrier()

    srcb = (srcb0, srcb1)
    dstb = (dstb0, dstb1)
    ridb = (ridb0, ridb1)
    exb = (exb0, exb1)
    dsc = (dsc0, dsc1)
    isems = (si0, si1)
    qsems = (sq0, sq1)
    rsems = (sr0, sr1)
    ssems = (ss0, ss1)
    ebase = sid * (EPAD // 16)

    def idx_copies(b, cc):
        sl = pl.ds(ebase + cc * KB, KB)
        return (pltpu.make_async_copy(srcp.at[sl], srcb[b], isems[b]),
                pltpu.make_async_copy(dstp.at[sl], dstb[b], isems[b]),
                pltpu.make_async_copy(ridp.at[sl], ridb[b], isems[b]),
                pltpu.make_async_copy(exp_.at[sl], exb[b], isems[b]))

    def qr_copy(b):
        return pltpu.make_async_copy(qr_sp.at[ridb[b]], stage.at[b],
                                     rsems[b])

    def scat_copy(b):
        return pltpu.make_async_copy(stage.at[b], a_sp.at[dsc[b]], ssems[b])

    def qs_start(b):
        @pl.when(cid == 0)
        def _():
            pltpu.make_async_copy(qlo.at[srcb[b]], rowsq.at[b],
                                  qsems[b]).start()

        @pl.when(cid == 1)
        def _():
            pltpu.make_async_copy(qhi.at[srcb[b]], rowsq.at[b],
                                  qsems[b]).start()

    def qs_wait(b):
        pltpu.make_async_copy(qlo.at[srcb[b]], rowsq.at[b], qsems[b]).wait()

    for cp in idx_copies(0, 0) + idx_copies(1, 1):
        cp.start()
    for cp in idx_copies(0, 0):
        cp.wait()
    qs_start(0)
    qr_copy(0).start()

    @pl.loop(0, NCB, step=2)
    def _chunk(c):
        for b in range(2):
            cc = c + b
            nb = 1 - b
            qs_wait(b)
            qr_copy(b).wait()

            @pl.when(cc + 1 < NCB)
            def _():
                for cp in idx_copies(nb, cc + 1):
                    cp.wait()

                @pl.when(cc >= 1)
                def _():
                    scat_copy(nb).wait()

                qs_start(nb)
                qr_copy(nb).start()

            @pl.loop(0, KB, unroll=4)
            def _edge(j):
                js = jnp.full((16,), j, dtype=jnp.int32)
                exs = plsc.load_gather(exb[b], [js])
                for cq in range(8):
                    sl = pl.ds(cq * 16, 16)
                    stage[b, j, sl] = (stage[b, j, sl]
                                       + rowsq[b, j, sl]) * exs

            for g in range(KB // 16):
                dsc[b][pl.ds(g * 16, 16)] = dstb[b][pl.ds(g * 16, 16)]
            scat_copy(b).start(add=True)

            @pl.when(cc + 2 < NCB)
            def _():
                for cp in idx_copies(b, cc + 2):
                    cp.start()

    scat_copy(0).wait()
    scat_copy(1).wait()
    plsc.subcore_barrier()
    pltpu.sync_copy(a_sp.at[pl.ds(sid * 640, 640)],
                    a2_out.at[cid, pl.ds(sid * 640, 640)])


def _final_body(a0_ref, a1_ref, d0_ref, d1_ref, o_ref):
    d = d0_ref[...] + d1_ref[...]
    ok = d > 0.0
    safe = jnp.where(ok, d, 1.0)
    o_ref[:, :128] = jnp.where(ok, a0_ref[...] / safe, 0.0)
    o_ref[:, 128:] = jnp.where(ok, a1_ref[...] / safe, 0.0)


def kernel(prev_h, emb_rel, edge_index, rid, pos_proj_w, attn_fc_w,
           weight_neighbor, k):
    f32 = jnp.float32
    i32 = jnp.int32

    wa = pos_proj_w[:D]
    wb = pos_proj_w[D:]
    wna = weight_neighbor[:D]
    wnb = weight_neighbor[D:]
    wcat = jnp.concatenate([wa, wb, wna], axis=1)

    pq = _tc_matmul(prev_h.astype(f32), wcat.astype(f32), 1000)
    qr = _tc_matmul(emb_rel.astype(f32), wnb.astype(f32), 200)

    zrows = jnp.zeros((NPAD - N, 128), f32)
    p1 = jnp.concatenate([pq[:, :D], jnp.zeros((NPAD - N, D), f32)], axis=0)
    p2 = jnp.concatenate([pq[:, D:2 * D], jnp.zeros((NPAD - N, D), f32)],
                         axis=0)
    qlo = jnp.concatenate([pq[:, 2 * D:2 * D + 128], zrows], axis=0)
    qhi = jnp.concatenate([pq[:, 2 * D + 128:], zrows], axis=0)
    zr = jnp.zeros((RPAD - NREL, 128), f32)
    qrcat = jnp.concatenate([qr[:, :128], zr, qr[:, 128:], zr], axis=0)

    pad = EPAD - E
    sent_n = (N + (jnp.arange(pad, dtype=i32) % 8)).astype(i32)
    sent_r = (NREL + (jnp.arange(pad, dtype=i32) % 8)).astype(i32)
    srcp = jnp.concatenate([edge_index[0].astype(i32), sent_n])
    dstp = jnp.concatenate([edge_index[1].astype(i32), sent_n])
    ridp = jnp.concatenate([rid.astype(i32), sent_r])
    wvec = attn_fc_w[:, 0].astype(f32)

    mesh = plsc.VectorSubcoreMesh(core_axis_name="c", subcore_axis_name="s")
    sc_params = pltpu.CompilerParams(needs_layout_passes=False)

    pass_a = functools.partial(
        pl.kernel,
        mesh=mesh,
        compiler_params=sc_params,
        out_type=[
            jax.ShapeDtypeStruct((EPAD // KA, KA), f32),
            jax.ShapeDtypeStruct((2, NPAD), f32),
        ],
        scratch_types=[
            pltpu.VMEM((KA,), i32),
            pltpu.VMEM((KA,), i32),
            pltpu.VMEM((KA,), i32),
            pltpu.VMEM((KA,), i32),
            pltpu.VMEM((KA,), i32),
            pltpu.VMEM((KA,), i32),
            pltpu.VMEM((2, KA, D), f32),
            pltpu.VMEM((2, KA, D), f32),
            pltpu.VMEM((NCA, KA), f32),
            pltpu.VMEM((KA,), f32),
            pltpu.VMEM((KA,), f32),
            pltpu.VMEM((D,), f32),
            pltpu.VMEM((640,), f32),
            pltpu.VMEM_SHARED((NPAD,), f32),
            pltpu.SemaphoreType.DMA,
            pltpu.SemaphoreType.DMA,
            pltpu.SemaphoreType.DMA,
            pltpu.SemaphoreType.DMA,
            pltpu.SemaphoreType.DMA,
            pltpu.SemaphoreType.DMA,
            pltpu.SemaphoreType.DMA,
            pltpu.SemaphoreType.DMA,
        ],
    )(_pass_a_body)
    ex2, d2 = pass_a(srcp, dstp, p1, p2, wvec)
    exflat = ex2.reshape(EPAD)

    pass_b = functools.partial(
        pl.kernel,
        mesh=mesh,
        compiler_params=sc_params,
        out_type=jax.ShapeDtypeStruct((2, NPAD, 128), f32),
        scratch_types=[
            pltpu.VMEM((KB,), i32),
            pltpu.VMEM((KB,), i32),
            pltpu.VMEM((KB,), i32),
            pltpu.VMEM((KB,), i32),
            pltpu.VMEM((KB,), i32),
            pltpu.VMEM((KB,), i32),
            pltpu.VMEM((KB,), f32),
            pltpu.VMEM((KB,), f32),
            pltpu.VMEM((KB,), i32),
            pltpu.VMEM((KB,), i32),
            pltpu.VMEM((2, KB, 128), f32),
            pltpu.VMEM((2, KB, 128), f32),
            pltpu.VMEM_SHARED((RPAD, 128), f32),
            pltpu.VMEM_SHARED((NPAD, 128), f32),
            pltpu.SemaphoreType.DMA,
            pltpu.SemaphoreType.DMA,
            pltpu.SemaphoreType.DMA,
            pltpu.SemaphoreType.DMA,
            pltpu.SemaphoreType.DMA,
            pltpu.SemaphoreType.DMA,
            pltpu.SemaphoreType.DMA,
            pltpu.SemaphoreType.DMA,
        ],
    )(_pass_b_body)
    a2 = pass_b(srcp, dstp, ridp, exflat, qlo, qhi, qrcat)

    out = pl.pallas_call(
        _final_body,
        grid=(8,),
        in_specs=[
            pl.BlockSpec((NPAD // 8, 128), lambda i: (i, 0)),
            pl.BlockSpec((NPAD // 8, 128), lambda i: (i, 0)),
            pl.BlockSpec((NPAD // 8, 1), lambda i: (i, 0)),
            pl.BlockSpec((NPAD // 8, 1), lambda i: (i, 0)),
        ],
        out_specs=pl.BlockSpec((NPAD // 8, D), lambda i: (i, 0)),
        out_shape=jax.ShapeDtypeStruct((NPAD, D), f32),
    )(a2[0], a2[1], d2[0].reshape(NPAD, 1), d2[1].reshape(NPAD, 1))

    return out[:N]
